# Initial kernel scaffold; baseline (speedup 1.0000x reference)
import jax
import jax.numpy as jnp
from jax.experimental import pallas as pl


def kernel(pos, batch, params):
    raise NotImplementedError("write your pallas kernel here")


# speedup vs baseline: 61.9786x; 61.9786x over previous
import functools

import jax
import jax.numpy as jnp
from jax import lax
from jax.experimental import pallas as pl
from jax.experimental.pallas import tpu as pltpu
from jax.experimental.pallas import tpu_sc as plsc

_CONV = [64, 64, 128, 256]
_K = 20
_REG = 1e-3
_B = 8
_PS = 1024
_BIG = 3e38


def _knn_body(posr_ref, post_ref, batr_ref, batt_ref, idx_ref, wn_ref, d2_ref):
    P, C = d2_ref.shape
    acc = jnp.zeros((P, C), jnp.float32)
    for c in range(3):
        xi = posr_ref[:, c:c + 1]
        xj = post_ref[c:c + 1, :]
        d = xi - xj
        acc = acc + d * d
    same = batr_ref[...] == batt_ref[...]
    d2_ref[...] = jnp.where(same, acc, jnp.full((P, C), _BIG, jnp.float32))
    cols = lax.broadcasted_iota(jnp.int32, (P, C), 1)
    vals = []
    for k in range(_K + 1):
        buf = d2_ref[...]
        m = jnp.min(buf, axis=1, keepdims=True)
        am = jnp.min(jnp.where(buf == m, cols, jnp.int32(C)), axis=1, keepdims=True)
        idx_ref[:, k:k + 1] = am
        vals.append(m)
        d2_ref[...] = jnp.where(cols == am, jnp.float32(jnp.inf), buf)
    w = [jnp.exp(-v) for v in vals[1:]]
    ws = functools.reduce(lambda a, b: a + b, w)
    inv = 1.0 / (ws + jnp.float32(_REG))
    wn_ref[:, 0:1] = jnp.zeros_like(inv)
    for k in range(_K):
        wn_ref[:, k + 1:k + 2] = w[k] * inv


def _knn(pos, batch):
    N = pos.shape[0]
    P = 128
    posT = pos.T
    bat_r = batch.reshape(N, 1)
    bat_t = batch.reshape(1, N)
    return pl.pallas_call(
        _knn_body,
        grid=(N // P,),
        in_specs=[
            pl.BlockSpec((P, 3), lambda i: (i, 0)),
            pl.BlockSpec((3, N), lambda i: (0, 0)),
            pl.BlockSpec((P, 1), lambda i: (i, 0)),
            pl.BlockSpec((1, N), lambda i: (0, 0)),
        ],
        out_specs=[
            pl.BlockSpec((P, _K + 1), lambda i: (i, 0)),
            pl.BlockSpec((P, _K + 1), lambda i: (i, 0)),
        ],
        out_shape=[
            jax.ShapeDtypeStruct((N, _K + 1), jnp.int32),
            jax.ShapeDtypeStruct((N, _K + 1), jnp.float32),
        ],
        scratch_shapes=[pltpu.VMEM((P, N), jnp.float32)],
    )(pos, posT, bat_r, bat_t)


def _gather_rows(table, idx):
    V, D = table.shape
    M = idx.shape[0]
    info = plsc.get_sparse_core_info()
    NC = info.num_cores
    NW = NC * info.num_subcores
    per_w = M // NW
    CH = 128
    n_ch = per_w // CH
    mesh = plsc.VectorSubcoreMesh(core_axis_name="c", subcore_axis_name="s")

    @functools.partial(
        pl.kernel,
        mesh=mesh,
        out_type=jax.ShapeDtypeStruct((M, D), jnp.float32),
        scratch_types=[
            pltpu.VMEM((CH,), jnp.int32),
            pltpu.VMEM((CH, D), jnp.float32),
            pltpu.SemaphoreType.DMA,
        ],
    )
    def gk(table_hbm, idx_hbm, out_hbm, idx_v, rows_v, sem):
        wid = lax.axis_index("s") * NC + lax.axis_index("c")
        base = wid * per_w

        def body(j, _):
            off = base + j * CH
            pltpu.sync_copy(idx_hbm.at[pl.ds(off, CH)], idx_v)
            pltpu.async_copy(table_hbm.at[idx_v], rows_v, sem).wait()
            pltpu.sync_copy(rows_v, out_hbm.at[pl.ds(off, CH)])
            return _

        lax.fori_loop(0, n_ch, body, 0)

    return gk(table, idx)


def _proj_body(x_ref, wd_ref, wb_ref, b0_ref, u_ref, v_ref):
    x = x_ref[...]
    u_ref[...] = jnp.dot(x, wd_ref[...], precision="highest",
                         preferred_element_type=jnp.float32) + b0_ref[...]
    v_ref[...] = jnp.dot(x, wb_ref[...], precision="highest",
                         preferred_element_type=jnp.float32)


def _proj(x, wd, wb, b0):
    N, ci = x.shape
    c = wd.shape[1]
    P = 512
    return pl.pallas_call(
        _proj_body,
        grid=(N // P,),
        in_specs=[
            pl.BlockSpec((P, ci), lambda i: (i, 0)),
            pl.BlockSpec((ci, c), lambda i: (0, 0)),
            pl.BlockSpec((ci, c), lambda i: (0, 0)),
            pl.BlockSpec((1, c), lambda i: (0, 0)),
        ],
        out_specs=[
            pl.BlockSpec((P, c), lambda i: (i, 0)),
            pl.BlockSpec((P, c), lambda i: (i, 0)),
        ],
        out_shape=[
            jax.ShapeDtypeStruct((N, c), jnp.float32),
            jax.ShapeDtypeStruct((N, c), jnp.float32),
        ],
    )(x, wd, wb, b0)


def _edge_body(u_ref, vg_ref, wn_ref, w1_ref, b1_ref, o_ref):
    u = u_ref[...]
    P, c = u.shape
    w1 = w1_ref[...]
    b1 = b1_ref[...]
    mx = jnp.full((P, c), -jnp.inf, jnp.float32)
    mn = jnp.zeros((P, c), jnp.float32)
    for k in range(_K):
        h1 = jnp.maximum(u + vg_ref[k][:, :c], 0.0)
        h2 = jnp.maximum(
            jnp.dot(h1, w1, precision="highest",
                    preferred_element_type=jnp.float32) + b1, 0.0)
        mx = jnp.maximum(mx, h2)
        mn = mn + wn_ref[:, k + 1:k + 2] * h2
    o_ref[...] = mx + mn


def _edge(u, vg, wn, w1, b1):
    N, c = u.shape
    Dp = vg.shape[-1]
    P = 256
    return pl.pallas_call(
        _edge_body,
        grid=(N // P,),
        in_specs=[
            pl.BlockSpec((P, c), lambda i: (i, 0)),
            pl.BlockSpec((_K, P, Dp), lambda i: (0, i, 0)),
            pl.BlockSpec((P, _K + 1), lambda i: (i, 0)),
            pl.BlockSpec((c, c), lambda i: (0, 0)),
            pl.BlockSpec((1, c), lambda i: (0, 0)),
        ],
        out_specs=pl.BlockSpec((P, c), lambda i: (i, 0)),
        out_shape=jax.ShapeDtypeStruct((N, c), jnp.float32),
    )(u, vg, wn, w1, b1)


def _pool_body(cat_ref, ew_ref, eb_ref, bat_ref, m_ref, s_ref):
    i = pl.program_id(0)
    e = jnp.maximum(
        jnp.dot(cat_ref[...], ew_ref[...], precision="highest",
                preferred_element_type=jnp.float32) + eb_ref[...], 0.0)

    @pl.when(i == 0)
    def _():
        m_ref[...] = jnp.full(m_ref.shape, -jnp.inf, jnp.float32)
        s_ref[...] = jnp.zeros(s_ref.shape, jnp.float32)

    bat = bat_ref[...]
    for b in range(_B):
        mb = bat == b
        m_ref[b:b + 1, :] = jnp.maximum(
            m_ref[b:b + 1, :],
            jnp.max(jnp.where(mb, e, -jnp.inf), axis=0, keepdims=True))
        s_ref[b:b + 1, :] = s_ref[b:b + 1, :] + jnp.sum(
            jnp.where(mb, e, 0.0), axis=0, keepdims=True)


def _pool(cat, ew, eb, bat):
    N, ci = cat.shape
    L = ew.shape[1]
    P = 512
    return pl.pallas_call(
        _pool_body,
        grid=(N // P,),
        in_specs=[
            pl.BlockSpec((P, ci), lambda i: (i, 0)),
            pl.BlockSpec((ci, L), lambda i: (0, 0)),
            pl.BlockSpec((1, L), lambda i: (0, 0)),
            pl.BlockSpec((P, 1), lambda i: (i, 0)),
        ],
        out_specs=[
            pl.BlockSpec((_B, L), lambda i: (0, 0)),
            pl.BlockSpec((_B, L), lambda i: (0, 0)),
        ],
        out_shape=[
            jax.ShapeDtypeStruct((_B, L), jnp.float32),
            jax.ShapeDtypeStruct((_B, L), jnp.float32),
        ],
    )(cat, ew, eb, bat)


def _head_body(g_ref, w0, b0, w1, b1, w2, b2, dw0, db0, dw1, db1, dw2, db2,
               o_ref):
    h = g_ref[...]
    layers = [(w0, b0, True), (w1, b1, True), (w2, b2, False),
              (dw0, db0, True), (dw1, db1, True), (dw2, db2, False)]
    for w, b, act in layers:
        h = jnp.dot(h, w[...], precision="highest",
                    preferred_element_type=jnp.float32) + b[...]
        if act:
            h = jnp.maximum(h, 0.0)
    o_ref[...] = h


def _head(g, p):
    names = ["cls_w0", "cls_b0", "cls_w1", "cls_b1", "cls_w2", "cls_b2",
             "dec_w0", "dec_b0", "dec_w1", "dec_b1", "dec_w2", "dec_b2"]
    args = []
    for n in names:
        a = p[n]
        args.append(a.reshape(1, -1) if a.ndim == 1 else a)
    return pl.pallas_call(
        _head_body,
        out_shape=jax.ShapeDtypeStruct((_B, 3 * _PS), jnp.float32),
    )(g, *args)


def kernel(pos, batch, params):
    N = pos.shape[0]
    batch = batch.astype(jnp.int32)
    idx, wn = _knn(pos, batch)
    nbr_flat = idx[:, 1:].T.reshape(-1)

    x = pos
    outs = []
    ci = 3
    for l, c in enumerate(_CONV):
        w0 = params[f"c{l}_w0"]
        wd = w0[:ci] - w0[ci:]
        wb = w0[ci:]
        u, v = _proj(x, wd, wb, params[f"c{l}_b0"].reshape(1, -1))
        if c < 128:
            v = jnp.pad(v, ((0, 0), (0, 128 - c)))
        vg = _gather_rows(v, nbr_flat).reshape(_K, N, v.shape[-1])
        x = _edge(u, vg, wn, params[f"c{l}_w1"], params[f"c{l}_b1"].reshape(1, -1))
        outs.append(x)
        ci = c

    cat = jnp.concatenate(outs, axis=-1)
    m, s = _pool(cat, params["emb_w"], params["emb_b"].reshape(1, -1),
                 batch.reshape(N, 1))
    cnt = jnp.sum((batch[:, None] == jnp.arange(_B)[None, :]).astype(jnp.float32),
                  axis=0)
    g = jnp.concatenate([m, s / cnt[:, None]], axis=-1)
    out = _head(g, params)
    return out.reshape(-1, _PS, 3)


# speedup vs baseline: 95.4301x; 1.5397x over previous
import functools

import jax
import jax.numpy as jnp
from jax import lax
from jax.experimental import pallas as pl
from jax.experimental.pallas import tpu as pltpu
from jax.experimental.pallas import tpu_sc as plsc

_CONV = [64, 64, 128, 256]
_K = 20
_REG = 1e-3
_B = 8
_PS = 1024
_BIG = 3e38


def _knn_body(co_ref, posr_ref, post_ref, batr_ref, batt_ref, idx_ref, wn_ref,
              d2_ref):
    P, W = d2_ref.shape
    co = pl.multiple_of(co_ref[pl.program_id(0)], 128)
    acc = jnp.zeros((P, W), jnp.float32)
    for c in range(3):
        xi = posr_ref[:, c:c + 1]
        xj = post_ref[c:c + 1, pl.ds(co, W)]
        d = xi - xj
        acc = acc + d * d
    same = batr_ref[...] == batt_ref[:, pl.ds(co, W)]
    d2_ref[...] = jnp.where(same, acc, jnp.full((P, W), _BIG, jnp.float32))
    cols = lax.broadcasted_iota(jnp.int32, (P, W), 1)
    vals = []
    for k in range(_K + 1):
        buf = d2_ref[...]
        m = jnp.min(buf, axis=1, keepdims=True)
        am = jnp.min(jnp.where(buf == m, cols, jnp.int32(W)), axis=1, keepdims=True)
        idx_ref[:, k:k + 1] = am + co
        vals.append(m)
        d2_ref[...] = jnp.where(cols == am, jnp.float32(jnp.inf), buf)
    w = [jnp.exp(-v) for v in vals[1:]]
    ws = functools.reduce(lambda a, b: a + b, w)
    inv = 1.0 / (ws + jnp.float32(_REG))
    wn_ref[:, 0:1] = jnp.zeros_like(inv)
    for k in range(_K):
        wn_ref[:, k + 1:k + 2] = w[k] * inv


_KNN_W = 2560


def _knn(pos, batch):
    N = pos.shape[0]
    P = 128
    nt = N // P
    posT = pos.T
    bat_r = batch.reshape(N, 1)
    bat_t = batch.reshape(1, N)

    starts = jnp.searchsorted(batch, jnp.arange(_B + 1, dtype=jnp.int32)
                              ).astype(jnp.int32)
    lo = starts[batch[::P]]
    hi = starts[batch[P - 1::P] + 1]
    win = min(_KNN_W, N)
    co = jnp.minimum((lo // 128) * 128, N - win)
    fits = jnp.max(hi - co) <= win

    def run(width, co_arr):
        grid_spec = pltpu.PrefetchScalarGridSpec(
            num_scalar_prefetch=1,
            grid=(nt,),
            in_specs=[
                pl.BlockSpec((P, 3), lambda i, s: (i, 0)),
                pl.BlockSpec((3, N), lambda i, s: (0, 0)),
                pl.BlockSpec((P, 1), lambda i, s: (i, 0)),
                pl.BlockSpec((1, N), lambda i, s: (0, 0)),
            ],
            out_specs=[
                pl.BlockSpec((P, _K + 1), lambda i, s: (i, 0)),
                pl.BlockSpec((P, _K + 1), lambda i, s: (i, 0)),
            ],
            scratch_shapes=[pltpu.VMEM((P, width), jnp.float32)],
        )
        return pl.pallas_call(
            _knn_body,
            grid_spec=grid_spec,
            out_shape=[
                jax.ShapeDtypeStruct((N, _K + 1), jnp.int32),
                jax.ShapeDtypeStruct((N, _K + 1), jnp.float32),
            ],
        )(co_arr, pos, posT, bat_r, bat_t)

    if win == N:
        return run(N, jnp.zeros((nt,), jnp.int32))
    return lax.cond(
        fits,
        lambda: run(win, co),
        lambda: run(N, jnp.zeros((nt,), jnp.int32)),
    )


def _gather_rows(table, idx):
    V, D = table.shape
    M = idx.shape[0]
    info = plsc.get_sparse_core_info()
    NC = info.num_cores
    NW = NC * info.num_subcores
    per_w = M // NW
    CH = 128
    n_ch = per_w // CH
    mesh = plsc.VectorSubcoreMesh(core_axis_name="c", subcore_axis_name="s")

    @functools.partial(
        pl.kernel,
        mesh=mesh,
        out_type=jax.ShapeDtypeStruct((M, D), jnp.float32),
        scratch_types=[
            pltpu.VMEM((CH,), jnp.int32),
            pltpu.VMEM((CH, D), jnp.float32),
            pltpu.SemaphoreType.DMA,
        ],
    )
    def gk(table_hbm, idx_hbm, out_hbm, idx_v, rows_v, sem):
        wid = lax.axis_index("s") * NC + lax.axis_index("c")
        base = wid * per_w

        def body(j, _):
            off = base + j * CH
            pltpu.sync_copy(idx_hbm.at[pl.ds(off, CH)], idx_v)
            pltpu.async_copy(table_hbm.at[idx_v], rows_v, sem).wait()
            pltpu.sync_copy(rows_v, out_hbm.at[pl.ds(off, CH)])
            return _

        lax.fori_loop(0, n_ch, body, 0)

    return gk(table, idx)


def _proj_body(x_ref, wd_ref, wb_ref, b0_ref, u_ref, v_ref):
    x = x_ref[...]
    u_ref[...] = jnp.dot(x, wd_ref[...], precision="highest",
                         preferred_element_type=jnp.float32) + b0_ref[...]
    v_ref[...] = jnp.dot(x, wb_ref[...], precision="highest",
                         preferred_element_type=jnp.float32)


def _proj(x, wd, wb, b0):
    N, ci = x.shape
    c = wd.shape[1]
    P = 512
    return pl.pallas_call(
        _proj_body,
        grid=(N // P,),
        in_specs=[
            pl.BlockSpec((P, ci), lambda i: (i, 0)),
            pl.BlockSpec((ci, c), lambda i: (0, 0)),
            pl.BlockSpec((ci, c), lambda i: (0, 0)),
            pl.BlockSpec((1, c), lambda i: (0, 0)),
        ],
        out_specs=[
            pl.BlockSpec((P, c), lambda i: (i, 0)),
            pl.BlockSpec((P, c), lambda i: (i, 0)),
        ],
        out_shape=[
            jax.ShapeDtypeStruct((N, c), jnp.float32),
            jax.ShapeDtypeStruct((N, c), jnp.float32),
        ],
    )(x, wd, wb, b0)


def _edge_body(u_ref, vg_ref, wn_ref, w1_ref, b1_ref, o_ref):
    P, c = u_ref.shape
    vg = vg_ref[...][:, :, :c]
    h1 = jnp.maximum(u_ref[...][None] + vg, 0.0)
    h2f = jnp.maximum(
        jnp.dot(h1.reshape(_K * P, c), w1_ref[...], precision="highest",
                preferred_element_type=jnp.float32) + b1_ref[...], 0.0)
    h2 = h2f.reshape(_K, P, c)
    mx = jnp.max(h2, axis=0)
    mn = jnp.sum(wn_ref[...] * h2, axis=0)
    o_ref[...] = mx + mn


def _edge(u, vg, wn3, w1, b1):
    N, c = u.shape
    Dp = vg.shape[-1]
    P = 256
    return pl.pallas_call(
        _edge_body,
        grid=(N // P,),
        in_specs=[
            pl.BlockSpec((P, c), lambda i: (i, 0)),
            pl.BlockSpec((_K, P, Dp), lambda i: (0, i, 0)),
            pl.BlockSpec((_K, P, 1), lambda i: (0, i, 0)),
            pl.BlockSpec((c, c), lambda i: (0, 0)),
            pl.BlockSpec((1, c), lambda i: (0, 0)),
        ],
        out_specs=pl.BlockSpec((P, c), lambda i: (i, 0)),
        out_shape=jax.ShapeDtypeStruct((N, c), jnp.float32),
    )(u, vg, wn3, w1, b1)


def _pool_body(cat_ref, ew_ref, eb_ref, bat_ref, m_ref, s_ref):
    i = pl.program_id(0)
    e = jnp.maximum(
        jnp.dot(cat_ref[...], ew_ref[...], precision="highest",
                preferred_element_type=jnp.float32) + eb_ref[...], 0.0)

    @pl.when(i == 0)
    def _():
        m_ref[...] = jnp.full(m_ref.shape, -jnp.inf, jnp.float32)
        s_ref[...] = jnp.zeros(s_ref.shape, jnp.float32)

    bat = bat_ref[...]
    for b in range(_B):
        mb = bat == b
        m_ref[b:b + 1, :] = jnp.maximum(
            m_ref[b:b + 1, :],
            jnp.max(jnp.where(mb, e, -jnp.inf), axis=0, keepdims=True))
        s_ref[b:b + 1, :] = s_ref[b:b + 1, :] + jnp.sum(
            jnp.where(mb, e, 0.0), axis=0, keepdims=True)


def _pool(cat, ew, eb, bat):
    N, ci = cat.shape
    L = ew.shape[1]
    P = 512
    return pl.pallas_call(
        _pool_body,
        grid=(N // P,),
        in_specs=[
            pl.BlockSpec((P, ci), lambda i: (i, 0)),
            pl.BlockSpec((ci, L), lambda i: (0, 0)),
            pl.BlockSpec((1, L), lambda i: (0, 0)),
            pl.BlockSpec((P, 1), lambda i: (i, 0)),
        ],
        out_specs=[
            pl.BlockSpec((_B, L), lambda i: (0, 0)),
            pl.BlockSpec((_B, L), lambda i: (0, 0)),
        ],
        out_shape=[
            jax.ShapeDtypeStruct((_B, L), jnp.float32),
            jax.ShapeDtypeStruct((_B, L), jnp.float32),
        ],
    )(cat, ew, eb, bat)


def _head_body(g_ref, w0, b0, w1, b1, w2, b2, dw0, db0, dw1, db1, dw2, db2,
               o_ref):
    h = g_ref[...]
    layers = [(w0, b0, True), (w1, b1, True), (w2, b2, False),
              (dw0, db0, True), (dw1, db1, True), (dw2, db2, False)]
    for w, b, act in layers:
        h = jnp.dot(h, w[...], precision="highest",
                    preferred_element_type=jnp.float32) + b[...]
        if act:
            h = jnp.maximum(h, 0.0)
    o_ref[...] = h


def _head(g, p):
    names = ["cls_w0", "cls_b0", "cls_w1", "cls_b1", "cls_w2", "cls_b2",
             "dec_w0", "dec_b0", "dec_w1", "dec_b1", "dec_w2", "dec_b2"]
    args = []
    for n in names:
        a = p[n]
        args.append(a.reshape(1, -1) if a.ndim == 1 else a)
    return pl.pallas_call(
        _head_body,
        out_shape=jax.ShapeDtypeStruct((_B, 3 * _PS), jnp.float32),
    )(g, *args)


def kernel(pos, batch, params):
    N = pos.shape[0]
    batch = batch.astype(jnp.int32)
    idx, wn = _knn(pos, batch)
    nbr_flat = idx[:, 1:].T.reshape(-1)
    wn3 = wn[:, 1:].T.reshape(_K, N, 1)

    x = pos
    outs = []
    ci = 3
    for l, c in enumerate(_CONV):
        w0 = params[f"c{l}_w0"]
        wd = w0[:ci] - w0[ci:]
        wb = w0[ci:]
        u, v = _proj(x, wd, wb, params[f"c{l}_b0"].reshape(1, -1))
        if c < 128:
            v = jnp.pad(v, ((0, 0), (0, 128 - c)))
        vg = _gather_rows(v, nbr_flat).reshape(_K, N, v.shape[-1])
        x = _edge(u, vg, wn3, params[f"c{l}_w1"], params[f"c{l}_b1"].reshape(1, -1))
        outs.append(x)
        ci = c

    cat = jnp.concatenate(outs, axis=-1)
    m, s = _pool(cat, params["emb_w"], params["emb_b"].reshape(1, -1),
                 batch.reshape(N, 1))
    cnt = jnp.sum((batch[:, None] == jnp.arange(_B)[None, :]).astype(jnp.float32),
                  axis=0)
    g = jnp.concatenate([m, s / cnt[:, None]], axis=-1)
    out = _head(g, params)
    return out.reshape(-1, _PS, 3)


# speedup vs baseline: 104.0499x; 1.0903x over previous
import functools

import jax
import jax.numpy as jnp
from jax import lax
from jax.experimental import pallas as pl
from jax.experimental.pallas import tpu as pltpu
from jax.experimental.pallas import tpu_sc as plsc

_CONV = [64, 64, 128, 256]
_K = 20
_REG = 1e-3
_B = 8
_PS = 1024
_BIG = 3e38


def _knn_body(co_ref, posr_ref, post_ref, batr_ref, batt_ref, idx_ref, wn_ref,
              d2_ref):
    P, W = d2_ref.shape
    co = pl.multiple_of(co_ref[pl.program_id(0)], 128)
    acc = jnp.zeros((P, W), jnp.float32)
    for c in range(3):
        xi = posr_ref[:, c:c + 1]
        xj = post_ref[c:c + 1, pl.ds(co, W)]
        d = xi - xj
        acc = acc + d * d
    same = batr_ref[...] == batt_ref[:, pl.ds(co, W)]
    d2_ref[...] = jnp.where(same, acc, jnp.full((P, W), _BIG, jnp.float32))
    cols = lax.broadcasted_iota(jnp.int32, (P, W), 1)
    vals = []
    for k in range(_K + 1):
        buf = d2_ref[...]
        m = jnp.min(buf, axis=1, keepdims=True)
        am = jnp.min(jnp.where(buf == m, cols, jnp.int32(W)), axis=1, keepdims=True)
        idx_ref[:, k:k + 1] = am + co
        vals.append(m)
        d2_ref[...] = jnp.where(cols == am, jnp.float32(jnp.inf), buf)
    w = [jnp.exp(-v) for v in vals[1:]]
    ws = functools.reduce(lambda a, b: a + b, w)
    inv = 1.0 / (ws + jnp.float32(_REG))
    wn_ref[:, 0:1] = jnp.zeros_like(inv)
    for k in range(_K):
        wn_ref[:, k + 1:k + 2] = w[k] * inv


_KNN_W = 2560


def _knn(pos, batch):
    N = pos.shape[0]
    P = 128
    nt = N // P
    posT = pos.T
    bat_r = batch.reshape(N, 1)
    bat_t = batch.reshape(1, N)

    starts = jnp.searchsorted(batch, jnp.arange(_B + 1, dtype=jnp.int32)
                              ).astype(jnp.int32)
    lo = starts[batch[::P]]
    hi = starts[batch[P - 1::P] + 1]
    win = min(_KNN_W, N)
    co = jnp.minimum((lo // 128) * 128, N - win)
    fits = jnp.max(hi - co) <= win

    def run(width, co_arr):
        grid_spec = pltpu.PrefetchScalarGridSpec(
            num_scalar_prefetch=1,
            grid=(nt,),
            in_specs=[
                pl.BlockSpec((P, 3), lambda i, s: (i, 0)),
                pl.BlockSpec((3, N), lambda i, s: (0, 0)),
                pl.BlockSpec((P, 1), lambda i, s: (i, 0)),
                pl.BlockSpec((1, N), lambda i, s: (0, 0)),
            ],
            out_specs=[
                pl.BlockSpec((P, _K + 1), lambda i, s: (i, 0)),
                pl.BlockSpec((P, _K + 1), lambda i, s: (i, 0)),
            ],
            scratch_shapes=[pltpu.VMEM((P, width), jnp.float32)],
        )
        return pl.pallas_call(
            _knn_body,
            grid_spec=grid_spec,
            out_shape=[
                jax.ShapeDtypeStruct((N, _K + 1), jnp.int32),
                jax.ShapeDtypeStruct((N, _K + 1), jnp.float32),
            ],
        )(co_arr, pos, posT, bat_r, bat_t)

    if win == N:
        return run(N, jnp.zeros((nt,), jnp.int32))
    return lax.cond(
        fits,
        lambda: run(win, co),
        lambda: run(N, jnp.zeros((nt,), jnp.int32)),
    )


def _gather_rows(table, idx):
    V, D = table.shape
    M = idx.shape[0]
    info = plsc.get_sparse_core_info()
    NC = info.num_cores
    NW = NC * info.num_subcores
    per_w = M // NW
    CH = 128
    n_ch = per_w // CH
    idx2 = idx.reshape(M // CH, CH)
    mesh = plsc.VectorSubcoreMesh(core_axis_name="c", subcore_axis_name="s")

    @functools.partial(
        pl.kernel,
        mesh=mesh,
        out_type=jax.ShapeDtypeStruct((M, D), jnp.float32),
        scratch_types=[
            pltpu.VMEM((n_ch, CH), jnp.int32),
            pltpu.VMEM((CH, D), jnp.float32),
            pltpu.VMEM((CH, D), jnp.float32),
            pltpu.SemaphoreType.DMA,
            pltpu.SemaphoreType.DMA,
            pltpu.SemaphoreType.DMA,
            pltpu.SemaphoreType.DMA,
        ],
    )
    def gk(table_hbm, idx_hbm, out_hbm, idx_v, rows0, rows1, g0, g1, s0, s1):
        wid = lax.axis_index("s") * NC + lax.axis_index("c")
        base = wid * per_w
        pltpu.sync_copy(idx_hbm.at[pl.ds(wid * n_ch, n_ch)], idx_v)

        def body(j, _):
            j0 = 2 * j
            j1 = 2 * j + 1
            cA = pltpu.async_copy(table_hbm.at[idx_v.at[j0]], rows0, g0)
            cB = pltpu.async_copy(table_hbm.at[idx_v.at[j1]], rows1, g1)
            cA.wait()
            sA = pltpu.async_copy(rows0, out_hbm.at[pl.ds(base + j0 * CH, CH)], s0)
            cB.wait()
            sB = pltpu.async_copy(rows1, out_hbm.at[pl.ds(base + j1 * CH, CH)], s1)
            sA.wait()
            sB.wait()
            return _

        lax.fori_loop(0, n_ch // 2, body, 0)

    return gk(table, idx2)


def _proj_body(x_ref, wd_ref, wb_ref, b0_ref, u_ref, v_ref):
    x = x_ref[...]
    u_ref[...] = jnp.dot(x, wd_ref[...], precision="highest",
                         preferred_element_type=jnp.float32) + b0_ref[...]
    v_ref[...] = jnp.dot(x, wb_ref[...], precision="highest",
                         preferred_element_type=jnp.float32)


def _proj(x, wd, wb, b0):
    N, ci = x.shape
    c = wd.shape[1]
    P = 512
    return pl.pallas_call(
        _proj_body,
        grid=(N // P,),
        in_specs=[
            pl.BlockSpec((P, ci), lambda i: (i, 0)),
            pl.BlockSpec((ci, c), lambda i: (0, 0)),
            pl.BlockSpec((ci, c), lambda i: (0, 0)),
            pl.BlockSpec((1, c), lambda i: (0, 0)),
        ],
        out_specs=[
            pl.BlockSpec((P, c), lambda i: (i, 0)),
            pl.BlockSpec((P, c), lambda i: (i, 0)),
        ],
        out_shape=[
            jax.ShapeDtypeStruct((N, c), jnp.float32),
            jax.ShapeDtypeStruct((N, c), jnp.float32),
        ],
    )(x, wd, wb, b0)


def _edge_body(u_ref, vg_ref, wn_ref, w1_ref, b1_ref, o_ref):
    P, c = u_ref.shape
    vg = vg_ref[...][:, :, :c]
    h1 = jnp.maximum(u_ref[...][None] + vg, 0.0)
    h2f = jnp.maximum(
        jnp.dot(h1.reshape(_K * P, c), w1_ref[...], precision="highest",
                preferred_element_type=jnp.float32) + b1_ref[...], 0.0)
    h2 = h2f.reshape(_K, P, c)
    mx = jnp.max(h2, axis=0)
    mn = jnp.sum(wn_ref[...] * h2, axis=0)
    o_ref[...] = mx + mn


def _edge(u, vg, wn3, w1, b1):
    N, c = u.shape
    Dp = vg.shape[-1]
    P = 256
    return pl.pallas_call(
        _edge_body,
        grid=(N // P,),
        in_specs=[
            pl.BlockSpec((P, c), lambda i: (i, 0)),
            pl.BlockSpec((_K, P, Dp), lambda i: (0, i, 0)),
            pl.BlockSpec((_K, P, 1), lambda i: (0, i, 0)),
            pl.BlockSpec((c, c), lambda i: (0, 0)),
            pl.BlockSpec((1, c), lambda i: (0, 0)),
        ],
        out_specs=pl.BlockSpec((P, c), lambda i: (i, 0)),
        out_shape=jax.ShapeDtypeStruct((N, c), jnp.float32),
    )(u, vg, wn3, w1, b1)


def _pool_body(cat_ref, ew_ref, eb_ref, bat_ref, m_ref, s_ref):
    i = pl.program_id(0)
    e = jnp.maximum(
        jnp.dot(cat_ref[...], ew_ref[...], precision="highest",
                preferred_element_type=jnp.float32) + eb_ref[...], 0.0)

    @pl.when(i == 0)
    def _():
        m_ref[...] = jnp.full(m_ref.shape, -jnp.inf, jnp.float32)
        s_ref[...] = jnp.zeros(s_ref.shape, jnp.float32)

    bat = bat_ref[...]
    for b in range(_B):
        mb = bat == b
        m_ref[b:b + 1, :] = jnp.maximum(
            m_ref[b:b + 1, :],
            jnp.max(jnp.where(mb, e, -jnp.inf), axis=0, keepdims=True))
        s_ref[b:b + 1, :] = s_ref[b:b + 1, :] + jnp.sum(
            jnp.where(mb, e, 0.0), axis=0, keepdims=True)


def _pool(cat, ew, eb, bat):
    N, ci = cat.shape
    L = ew.shape[1]
    P = 512
    return pl.pallas_call(
        _pool_body,
        grid=(N // P,),
        in_specs=[
            pl.BlockSpec((P, ci), lambda i: (i, 0)),
            pl.BlockSpec((ci, L), lambda i: (0, 0)),
            pl.BlockSpec((1, L), lambda i: (0, 0)),
            pl.BlockSpec((P, 1), lambda i: (i, 0)),
        ],
        out_specs=[
            pl.BlockSpec((_B, L), lambda i: (0, 0)),
            pl.BlockSpec((_B, L), lambda i: (0, 0)),
        ],
        out_shape=[
            jax.ShapeDtypeStruct((_B, L), jnp.float32),
            jax.ShapeDtypeStruct((_B, L), jnp.float32),
        ],
    )(cat, ew, eb, bat)


def _head_body(g_ref, w0, b0, w1, b1, w2, b2, dw0, db0, dw1, db1, dw2, db2,
               o_ref):
    h = g_ref[...]
    layers = [(w0, b0, True), (w1, b1, True), (w2, b2, False),
              (dw0, db0, True), (dw1, db1, True), (dw2, db2, False)]
    for w, b, act in layers:
        h = jnp.dot(h, w[...], precision="highest",
                    preferred_element_type=jnp.float32) + b[...]
        if act:
            h = jnp.maximum(h, 0.0)
    o_ref[...] = h


def _head(g, p):
    names = ["cls_w0", "cls_b0", "cls_w1", "cls_b1", "cls_w2", "cls_b2",
             "dec_w0", "dec_b0", "dec_w1", "dec_b1", "dec_w2", "dec_b2"]
    args = []
    for n in names:
        a = p[n]
        args.append(a.reshape(1, -1) if a.ndim == 1 else a)
    return pl.pallas_call(
        _head_body,
        out_shape=jax.ShapeDtypeStruct((_B, 3 * _PS), jnp.float32),
    )(g, *args)


def kernel(pos, batch, params):
    N = pos.shape[0]
    batch = batch.astype(jnp.int32)
    idx, wn = _knn(pos, batch)
    nbr_flat = idx[:, 1:].T.reshape(-1)
    wn3 = wn[:, 1:].T.reshape(_K, N, 1)

    x = pos
    outs = []
    ci = 3
    for l, c in enumerate(_CONV):
        w0 = params[f"c{l}_w0"]
        wd = w0[:ci] - w0[ci:]
        wb = w0[ci:]
        u, v = _proj(x, wd, wb, params[f"c{l}_b0"].reshape(1, -1))
        if c < 128:
            v = jnp.pad(v, ((0, 0), (0, 128 - c)))
        vg = _gather_rows(v, nbr_flat).reshape(_K, N, v.shape[-1])
        x = _edge(u, vg, wn3, params[f"c{l}_w1"], params[f"c{l}_b1"].reshape(1, -1))
        outs.append(x)
        ci = c

    cat = jnp.concatenate(outs, axis=-1)
    m, s = _pool(cat, params["emb_w"], params["emb_b"].reshape(1, -1),
                 batch.reshape(N, 1))
    cnt = jnp.sum((batch[:, None] == jnp.arange(_B)[None, :]).astype(jnp.float32),
                  axis=0)
    g = jnp.concatenate([m, s / cnt[:, None]], axis=-1)
    out = _head(g, params)
    return out.reshape(-1, _PS, 3)


# speedup vs baseline: 111.8120x; 1.0746x over previous
import functools

import jax
import jax.numpy as jnp
from jax import lax
from jax.experimental import pallas as pl
from jax.experimental.pallas import tpu as pltpu
from jax.experimental.pallas import tpu_sc as plsc

_CONV = [64, 64, 128, 256]
_K = 20
_REG = 1e-3
_B = 8
_PS = 1024
_BIG = 3e38


def _knn_extract(co, posr_ref, post_ref, batr_ref, batt_ref, d2_ref):
    P, W = d2_ref.shape
    acc = jnp.zeros((P, W), jnp.float32)
    for c in range(3):
        xi = posr_ref[:, c:c + 1]
        xj = post_ref[c:c + 1, pl.ds(co, W)]
        d = xi - xj
        acc = acc + d * d
    same = batr_ref[...] == batt_ref[:, pl.ds(co, W)]
    d2_ref[...] = jnp.where(same, acc, jnp.full((P, W), _BIG, jnp.float32))
    cols = lax.broadcasted_iota(jnp.int32, (P, W), 1)
    vals, idxs = [], []
    for k in range(_K + 1):
        buf = d2_ref[...]
        m = jnp.min(buf, axis=1, keepdims=True)
        am = jnp.min(jnp.where(buf == m, cols, jnp.int32(W)), axis=1, keepdims=True)
        idxs.append(am + co)
        vals.append(m)
        d2_ref[...] = jnp.where(cols == am, jnp.float32(jnp.inf), buf)
    w = [jnp.exp(-v) for v in vals[1:]]
    ws = functools.reduce(lambda a, b: a + b, w)
    inv = 1.0 / (ws + jnp.float32(_REG))
    wcols = [jnp.zeros_like(inv)] + [wk * inv for wk in w]
    return jnp.concatenate(idxs, axis=1), jnp.concatenate(wcols, axis=1)


def _knn_body(coa_ref, cob_ref, posr_ref, post_ref, batr_ref, batt_ref,
              idx_ref, wn_ref, d2_ref):
    i = pl.program_id(0)
    co_a = pl.multiple_of(coa_ref[i], 128)
    co_b = pl.multiple_of(cob_ref[i], 128)
    idx_a, wn_a = _knn_extract(co_a, posr_ref, post_ref, batr_ref, batt_ref,
                               d2_ref)
    straddle = co_b != co_a

    @pl.when(straddle)
    def _():
        idx_b, wn_b = _knn_extract(co_b, posr_ref, post_ref, batr_ref,
                                   batt_ref, d2_ref)
        rm = batr_ref[...] == batr_ref[0:1, :]
        idx_ref[...] = jnp.where(rm, idx_a, idx_b)
        wn_ref[...] = jnp.where(rm, wn_a, wn_b)

    @pl.when(jnp.logical_not(straddle))
    def _():
        idx_ref[...] = idx_a
        wn_ref[...] = wn_a


_KNN_W = 1408


def _knn(pos, batch):
    N = pos.shape[0]
    P = 128
    nt = N // P
    posT = pos.T
    bat_r = batch.reshape(N, 1)
    bat_t = batch.reshape(1, N)

    starts = jnp.searchsorted(batch, jnp.arange(_B + 1, dtype=jnp.int32)
                              ).astype(jnp.int32)
    win = min(_KNN_W, N)
    cpc = jnp.minimum((starts[:-1] // 128) * 128, N - win)
    co_a = cpc[batch[::P]]
    co_b = cpc[batch[P - 1::P]]
    sz = starts[1:] - starts[:-1]
    fits = (jnp.max(starts[1:] - cpc) <= win) & (jnp.min(sz) >= P)

    def run(width, ca, cb):
        grid_spec = pltpu.PrefetchScalarGridSpec(
            num_scalar_prefetch=2,
            grid=(nt,),
            in_specs=[
                pl.BlockSpec((P, 3), lambda i, s1, s2: (i, 0)),
                pl.BlockSpec((3, N), lambda i, s1, s2: (0, 0)),
                pl.BlockSpec((P, 1), lambda i, s1, s2: (i, 0)),
                pl.BlockSpec((1, N), lambda i, s1, s2: (0, 0)),
            ],
            out_specs=[
                pl.BlockSpec((P, _K + 1), lambda i, s1, s2: (i, 0)),
                pl.BlockSpec((P, _K + 1), lambda i, s1, s2: (i, 0)),
            ],
            scratch_shapes=[pltpu.VMEM((P, width), jnp.float32)],
        )
        return pl.pallas_call(
            _knn_body,
            grid_spec=grid_spec,
            out_shape=[
                jax.ShapeDtypeStruct((N, _K + 1), jnp.int32),
                jax.ShapeDtypeStruct((N, _K + 1), jnp.float32),
            ],
        )(ca, cb, pos, posT, bat_r, bat_t)

    zeros = jnp.zeros((nt,), jnp.int32)
    if win == N:
        return run(N, zeros, zeros)
    return lax.cond(
        fits,
        lambda: run(win, co_a, co_b),
        lambda: run(N, zeros, zeros),
    )


def _gather_rows(table, idx):
    V, D = table.shape
    M = idx.shape[0]
    info = plsc.get_sparse_core_info()
    NC = info.num_cores
    NW = NC * info.num_subcores
    per_w = M // NW
    CH = 128
    n_ch = per_w // CH
    idx2 = idx.reshape(M // CH, CH)
    mesh = plsc.VectorSubcoreMesh(core_axis_name="c", subcore_axis_name="s")

    @functools.partial(
        pl.kernel,
        mesh=mesh,
        out_type=jax.ShapeDtypeStruct((M, D), jnp.float32),
        scratch_types=[
            pltpu.VMEM((n_ch, CH), jnp.int32),
            pltpu.VMEM((CH, D), jnp.float32),
            pltpu.VMEM((CH, D), jnp.float32),
            pltpu.SemaphoreType.DMA,
            pltpu.SemaphoreType.DMA,
            pltpu.SemaphoreType.DMA,
            pltpu.SemaphoreType.DMA,
        ],
    )
    def gk(table_hbm, idx_hbm, out_hbm, idx_v, rows0, rows1, g0, g1, s0, s1):
        wid = lax.axis_index("s") * NC + lax.axis_index("c")
        base = wid * per_w
        pltpu.sync_copy(idx_hbm.at[pl.ds(wid * n_ch, n_ch)], idx_v)

        def body(j, _):
            j0 = 2 * j
            j1 = 2 * j + 1
            cA = pltpu.async_copy(table_hbm.at[idx_v.at[j0]], rows0, g0)
            cB = pltpu.async_copy(table_hbm.at[idx_v.at[j1]], rows1, g1)
            cA.wait()
            sA = pltpu.async_copy(rows0, out_hbm.at[pl.ds(base + j0 * CH, CH)], s0)
            cB.wait()
            sB = pltpu.async_copy(rows1, out_hbm.at[pl.ds(base + j1 * CH, CH)], s1)
            sA.wait()
            sB.wait()
            return _

        lax.fori_loop(0, n_ch // 2, body, 0)

    return gk(table, idx2)


def _proj_body(x_ref, wd_ref, wb_ref, b0_ref, u_ref, v_ref):
    x = x_ref[...]
    u_ref[...] = jnp.dot(x, wd_ref[...], precision="highest",
                         preferred_element_type=jnp.float32) + b0_ref[...]
    v_ref[...] = jnp.dot(x, wb_ref[...], precision="highest",
                         preferred_element_type=jnp.float32)


def _proj(x, wd, wb, b0):
    N, ci = x.shape
    c = wd.shape[1]
    P = 512
    return pl.pallas_call(
        _proj_body,
        grid=(N // P,),
        in_specs=[
            pl.BlockSpec((P, ci), lambda i: (i, 0)),
            pl.BlockSpec((ci, c), lambda i: (0, 0)),
            pl.BlockSpec((ci, c), lambda i: (0, 0)),
            pl.BlockSpec((1, c), lambda i: (0, 0)),
        ],
        out_specs=[
            pl.BlockSpec((P, c), lambda i: (i, 0)),
            pl.BlockSpec((P, c), lambda i: (i, 0)),
        ],
        out_shape=[
            jax.ShapeDtypeStruct((N, c), jnp.float32),
            jax.ShapeDtypeStruct((N, c), jnp.float32),
        ],
    )(x, wd, wb, b0)


def _edge_body(u_ref, vg_ref, wn_ref, w1_ref, b1_ref, o_ref):
    P, c = u_ref.shape
    vg = vg_ref[...][:, :, :c]
    h1 = jnp.maximum(u_ref[...][None] + vg, 0.0)
    h2f = jnp.maximum(
        jnp.dot(h1.reshape(_K * P, c), w1_ref[...], precision="highest",
                preferred_element_type=jnp.float32) + b1_ref[...], 0.0)
    h2 = h2f.reshape(_K, P, c)
    mx = jnp.max(h2, axis=0)
    mn = jnp.sum(wn_ref[...] * h2, axis=0)
    o_ref[...] = mx + mn


def _edge(u, vg, wn3, w1, b1):
    N, c = u.shape
    Dp = vg.shape[-1]
    P = 256
    return pl.pallas_call(
        _edge_body,
        grid=(N // P,),
        in_specs=[
            pl.BlockSpec((P, c), lambda i: (i, 0)),
            pl.BlockSpec((_K, P, Dp), lambda i: (0, i, 0)),
            pl.BlockSpec((_K, P, 1), lambda i: (0, i, 0)),
            pl.BlockSpec((c, c), lambda i: (0, 0)),
            pl.BlockSpec((1, c), lambda i: (0, 0)),
        ],
        out_specs=pl.BlockSpec((P, c), lambda i: (i, 0)),
        out_shape=jax.ShapeDtypeStruct((N, c), jnp.float32),
    )(u, vg, wn3, w1, b1)


def _pool_body(cat_ref, ew_ref, eb_ref, bat_ref, m_ref, s_ref):
    i = pl.program_id(0)
    e = jnp.maximum(
        jnp.dot(cat_ref[...], ew_ref[...], precision="highest",
                preferred_element_type=jnp.float32) + eb_ref[...], 0.0)

    @pl.when(i == 0)
    def _():
        m_ref[...] = jnp.full(m_ref.shape, -jnp.inf, jnp.float32)
        s_ref[...] = jnp.zeros(s_ref.shape, jnp.float32)

    bat = bat_ref[...]
    for b in range(_B):
        mb = bat == b
        m_ref[b:b + 1, :] = jnp.maximum(
            m_ref[b:b + 1, :],
            jnp.max(jnp.where(mb, e, -jnp.inf), axis=0, keepdims=True))
        s_ref[b:b + 1, :] = s_ref[b:b + 1, :] + jnp.sum(
            jnp.where(mb, e, 0.0), axis=0, keepdims=True)


def _pool(cat, ew, eb, bat):
    N, ci = cat.shape
    L = ew.shape[1]
    P = 512
    return pl.pallas_call(
        _pool_body,
        grid=(N // P,),
        in_specs=[
            pl.BlockSpec((P, ci), lambda i: (i, 0)),
            pl.BlockSpec((ci, L), lambda i: (0, 0)),
            pl.BlockSpec((1, L), lambda i: (0, 0)),
            pl.BlockSpec((P, 1), lambda i: (i, 0)),
        ],
        out_specs=[
            pl.BlockSpec((_B, L), lambda i: (0, 0)),
            pl.BlockSpec((_B, L), lambda i: (0, 0)),
        ],
        out_shape=[
            jax.ShapeDtypeStruct((_B, L), jnp.float32),
            jax.ShapeDtypeStruct((_B, L), jnp.float32),
        ],
    )(cat, ew, eb, bat)


def _head_body(g_ref, w0, b0, w1, b1, w2, b2, dw0, db0, dw1, db1, dw2, db2,
               o_ref):
    h = g_ref[...]
    layers = [(w0, b0, True), (w1, b1, True), (w2, b2, False),
              (dw0, db0, True), (dw1, db1, True), (dw2, db2, False)]
    for w, b, act in layers:
        h = jnp.dot(h, w[...], precision="highest",
                    preferred_element_type=jnp.float32) + b[...]
        if act:
            h = jnp.maximum(h, 0.0)
    o_ref[...] = h


def _head(g, p):
    names = ["cls_w0", "cls_b0", "cls_w1", "cls_b1", "cls_w2", "cls_b2",
             "dec_w0", "dec_b0", "dec_w1", "dec_b1", "dec_w2", "dec_b2"]
    args = []
    for n in names:
        a = p[n]
        args.append(a.reshape(1, -1) if a.ndim == 1 else a)
    return pl.pallas_call(
        _head_body,
        out_shape=jax.ShapeDtypeStruct((_B, 3 * _PS), jnp.float32),
    )(g, *args)


def kernel(pos, batch, params):
    N = pos.shape[0]
    batch = batch.astype(jnp.int32)
    idx, wn = _knn(pos, batch)
    nbr_flat = idx[:, 1:].T.reshape(-1)
    wn3 = wn[:, 1:].T.reshape(_K, N, 1)

    x = pos
    outs = []
    ci = 3
    for l, c in enumerate(_CONV):
        w0 = params[f"c{l}_w0"]
        wd = w0[:ci] - w0[ci:]
        wb = w0[ci:]
        u, v = _proj(x, wd, wb, params[f"c{l}_b0"].reshape(1, -1))
        if c < 128:
            v = jnp.pad(v, ((0, 0), (0, 128 - c)))
        vg = _gather_rows(v, nbr_flat).reshape(_K, N, v.shape[-1])
        x = _edge(u, vg, wn3, params[f"c{l}_w1"], params[f"c{l}_b1"].reshape(1, -1))
        outs.append(x)
        ci = c

    cat = jnp.concatenate(outs, axis=-1)
    m, s = _pool(cat, params["emb_w"], params["emb_b"].reshape(1, -1),
                 batch.reshape(N, 1))
    cnt = jnp.sum((batch[:, None] == jnp.arange(_B)[None, :]).astype(jnp.float32),
                  axis=0)
    g = jnp.concatenate([m, s / cnt[:, None]], axis=-1)
    out = _head(g, params)
    return out.reshape(-1, _PS, 3)


# speedup vs baseline: 129.9671x; 1.1624x over previous
import functools

import jax
import jax.numpy as jnp
from jax import lax
from jax.experimental import pallas as pl
from jax.experimental.pallas import tpu as pltpu
from jax.experimental.pallas import tpu_sc as plsc

_CONV = [64, 64, 128, 256]
_K = 20
_REG = 1e-3
_B = 8
_PS = 1024
_BIG = 3e38


def _knn_extract(co, posr_ref, post_ref, batr_ref, batt_ref, d2_ref):
    P, W = d2_ref.shape
    acc = jnp.zeros((P, W), jnp.float32)
    for c in range(3):
        xi = posr_ref[:, c:c + 1]
        xj = post_ref[c:c + 1, pl.ds(co, W)]
        d = xi - xj
        acc = acc + d * d
    same = batr_ref[...] == batt_ref[:, pl.ds(co, W)]
    d2_ref[...] = jnp.where(same, acc, jnp.full((P, W), _BIG, jnp.float32))
    cols = lax.broadcasted_iota(jnp.int32, (P, W), 1)
    vals, idxs = [], []
    for k in range(_K + 1):
        buf = d2_ref[...]
        m = jnp.min(buf, axis=1, keepdims=True)
        am = jnp.min(jnp.where(buf == m, cols, jnp.int32(W)), axis=1, keepdims=True)
        idxs.append(am + co)
        vals.append(m)
        d2_ref[...] = jnp.where(cols == am, jnp.float32(jnp.inf), buf)
    w = [jnp.exp(-v) for v in vals[1:]]
    ws = functools.reduce(lambda a, b: a + b, w)
    inv = 1.0 / (ws + jnp.float32(_REG))
    wcols = [jnp.zeros_like(inv)] + [wk * inv for wk in w]
    return jnp.concatenate(idxs, axis=1), jnp.concatenate(wcols, axis=1)


def _knn_body(coa_ref, cob_ref, posr_ref, post_ref, batr_ref, batt_ref,
              idx_ref, wn_ref, d2_ref):
    i = pl.program_id(0)
    co_a = pl.multiple_of(coa_ref[i], 128)
    co_b = pl.multiple_of(cob_ref[i], 128)
    idx_a, wn_a = _knn_extract(co_a, posr_ref, post_ref, batr_ref, batt_ref,
                               d2_ref)
    straddle = co_b != co_a

    @pl.when(straddle)
    def _():
        idx_b, wn_b = _knn_extract(co_b, posr_ref, post_ref, batr_ref,
                                   batt_ref, d2_ref)
        rm = batr_ref[...] == batr_ref[0:1, :]
        idx_ref[...] = jnp.where(rm, idx_a, idx_b)
        wn_ref[...] = jnp.where(rm, wn_a, wn_b)

    @pl.when(jnp.logical_not(straddle))
    def _():
        idx_ref[...] = idx_a
        wn_ref[...] = wn_a


_KNN_W = 1408


def _knn(pos, batch):
    N = pos.shape[0]
    P = 128
    nt = N // P
    posT = pos.T
    bat_r = batch.reshape(N, 1)
    bat_t = batch.reshape(1, N)

    starts = jnp.searchsorted(batch, jnp.arange(_B + 1, dtype=jnp.int32)
                              ).astype(jnp.int32)
    win = min(_KNN_W, N)
    cpc = jnp.minimum((starts[:-1] // 128) * 128, N - win)
    co_a = cpc[batch[::P]]
    co_b = cpc[batch[P - 1::P]]
    sz = starts[1:] - starts[:-1]
    fits = (jnp.max(starts[1:] - cpc) <= win) & (jnp.min(sz) >= P)

    def run(width, ca, cb):
        grid_spec = pltpu.PrefetchScalarGridSpec(
            num_scalar_prefetch=2,
            grid=(nt,),
            in_specs=[
                pl.BlockSpec((P, 3), lambda i, s1, s2: (i, 0)),
                pl.BlockSpec((3, N), lambda i, s1, s2: (0, 0)),
                pl.BlockSpec((P, 1), lambda i, s1, s2: (i, 0)),
                pl.BlockSpec((1, N), lambda i, s1, s2: (0, 0)),
            ],
            out_specs=[
                pl.BlockSpec((P, _K + 1), lambda i, s1, s2: (i, 0)),
                pl.BlockSpec((P, _K + 1), lambda i, s1, s2: (i, 0)),
            ],
            scratch_shapes=[pltpu.VMEM((P, width), jnp.float32)],
        )
        return pl.pallas_call(
            _knn_body,
            grid_spec=grid_spec,
            out_shape=[
                jax.ShapeDtypeStruct((N, _K + 1), jnp.int32),
                jax.ShapeDtypeStruct((N, _K + 1), jnp.float32),
            ],
        )(ca, cb, pos, posT, bat_r, bat_t)

    zeros = jnp.zeros((nt,), jnp.int32)
    if win == N:
        return run(N, zeros, zeros)
    return lax.cond(
        fits,
        lambda: run(win, co_a, co_b),
        lambda: run(N, zeros, zeros),
    )


def _gather_rows(table, idx):
    V, D = table.shape
    M = idx.shape[0]
    info = plsc.get_sparse_core_info()
    NC = info.num_cores
    NW = NC * info.num_subcores
    per_w = M // NW
    CH = 128
    n_ch = per_w // CH
    idx2 = idx.reshape(M // CH, CH)
    mesh = plsc.VectorSubcoreMesh(core_axis_name="c", subcore_axis_name="s")

    @functools.partial(
        pl.kernel,
        mesh=mesh,
        out_type=jax.ShapeDtypeStruct((M, D), jnp.float32),
        scratch_types=[
            pltpu.VMEM((n_ch, CH), jnp.int32),
            pltpu.VMEM((CH, D), jnp.float32),
            pltpu.VMEM((CH, D), jnp.float32),
            pltpu.SemaphoreType.DMA,
            pltpu.SemaphoreType.DMA,
            pltpu.SemaphoreType.DMA,
            pltpu.SemaphoreType.DMA,
        ],
    )
    def gk(table_hbm, idx_hbm, out_hbm, idx_v, rows0, rows1, g0, g1, s0, s1):
        wid = lax.axis_index("s") * NC + lax.axis_index("c")
        base = wid * per_w
        pltpu.sync_copy(idx_hbm.at[pl.ds(wid * n_ch, n_ch)], idx_v)

        def body(j, _):
            j0 = 2 * j
            j1 = 2 * j + 1
            cA = pltpu.async_copy(table_hbm.at[idx_v.at[j0]], rows0, g0)
            cB = pltpu.async_copy(table_hbm.at[idx_v.at[j1]], rows1, g1)
            cA.wait()
            sA = pltpu.async_copy(rows0, out_hbm.at[pl.ds(base + j0 * CH, CH)], s0)
            cB.wait()
            sB = pltpu.async_copy(rows1, out_hbm.at[pl.ds(base + j1 * CH, CH)], s1)
            sA.wait()
            sB.wait()
            return _

        lax.fori_loop(0, n_ch // 2, body, 0)

    return gk(table, idx2)


def _proj_body(x_ref, wd_ref, wb_ref, b0_ref, u_ref, v_ref):
    x = x_ref[...]
    u_ref[...] = jnp.dot(x, wd_ref[...],
                         preferred_element_type=jnp.float32) + b0_ref[...]
    v_ref[...] = jnp.dot(x, wb_ref[...],
                         preferred_element_type=jnp.float32)


def _proj(x, wd, wb, b0):
    N, ci = x.shape
    c = wd.shape[1]
    P = 512
    return pl.pallas_call(
        _proj_body,
        grid=(N // P,),
        in_specs=[
            pl.BlockSpec((P, ci), lambda i: (i, 0)),
            pl.BlockSpec((ci, c), lambda i: (0, 0)),
            pl.BlockSpec((ci, c), lambda i: (0, 0)),
            pl.BlockSpec((1, c), lambda i: (0, 0)),
        ],
        out_specs=[
            pl.BlockSpec((P, c), lambda i: (i, 0)),
            pl.BlockSpec((P, c), lambda i: (i, 0)),
        ],
        out_shape=[
            jax.ShapeDtypeStruct((N, c), jnp.float32),
            jax.ShapeDtypeStruct((N, c), jnp.float32),
        ],
    )(x, wd, wb, b0)


def _edge_body(u_ref, vg_ref, wn_ref, w1_ref, b1_ref, o_ref):
    P, c = u_ref.shape
    vg = vg_ref[...][:, :, :c]
    h1 = jnp.maximum(u_ref[...][None] + vg, 0.0)
    h2f = jnp.maximum(
        jnp.dot(h1.reshape(_K * P, c), w1_ref[...],
                preferred_element_type=jnp.float32) + b1_ref[...], 0.0)
    h2 = h2f.reshape(_K, P, c)
    mx = jnp.max(h2, axis=0)
    mn = jnp.sum(wn_ref[...] * h2, axis=0)
    o_ref[...] = mx + mn


def _edge(u, vg, wn3, w1, b1):
    N, c = u.shape
    Dp = vg.shape[-1]
    P = 256
    return pl.pallas_call(
        _edge_body,
        grid=(N // P,),
        in_specs=[
            pl.BlockSpec((P, c), lambda i: (i, 0)),
            pl.BlockSpec((_K, P, Dp), lambda i: (0, i, 0)),
            pl.BlockSpec((_K, P, 1), lambda i: (0, i, 0)),
            pl.BlockSpec((c, c), lambda i: (0, 0)),
            pl.BlockSpec((1, c), lambda i: (0, 0)),
        ],
        out_specs=pl.BlockSpec((P, c), lambda i: (i, 0)),
        out_shape=jax.ShapeDtypeStruct((N, c), jnp.float32),
    )(u, vg, wn3, w1, b1)


def _pool_body(cat_ref, ew_ref, eb_ref, bat_ref, m_ref, s_ref):
    i = pl.program_id(0)
    e = jnp.maximum(
        jnp.dot(cat_ref[...], ew_ref[...],
                preferred_element_type=jnp.float32) + eb_ref[...], 0.0)

    @pl.when(i == 0)
    def _():
        m_ref[...] = jnp.full(m_ref.shape, -jnp.inf, jnp.float32)
        s_ref[...] = jnp.zeros(s_ref.shape, jnp.float32)

    bat = bat_ref[...]
    oh = (bat == lax.broadcasted_iota(jnp.int32, (1, _B), 1)).astype(jnp.float32)
    s_ref[...] = s_ref[...] + lax.dot_general(
        oh, e, (((0,), (0,)), ((), ())), precision="highest",
        preferred_element_type=jnp.float32)
    for b in range(_B):
        mb = bat == b
        m_ref[b:b + 1, :] = jnp.maximum(
            m_ref[b:b + 1, :],
            jnp.max(jnp.where(mb, e, -jnp.inf), axis=0, keepdims=True))


def _pool(cat, ew, eb, bat):
    N, ci = cat.shape
    L = ew.shape[1]
    P = 512
    return pl.pallas_call(
        _pool_body,
        grid=(N // P,),
        in_specs=[
            pl.BlockSpec((P, ci), lambda i: (i, 0)),
            pl.BlockSpec((ci, L), lambda i: (0, 0)),
            pl.BlockSpec((1, L), lambda i: (0, 0)),
            pl.BlockSpec((P, 1), lambda i: (i, 0)),
        ],
        out_specs=[
            pl.BlockSpec((_B, L), lambda i: (0, 0)),
            pl.BlockSpec((_B, L), lambda i: (0, 0)),
        ],
        out_shape=[
            jax.ShapeDtypeStruct((_B, L), jnp.float32),
            jax.ShapeDtypeStruct((_B, L), jnp.float32),
        ],
    )(cat, ew, eb, bat)


def _head_body(g_ref, w0, b0, w1, b1, w2, b2, dw0, db0, dw1, db1, dw2, db2,
               o_ref):
    h = g_ref[...]
    layers = [(w0, b0, True), (w1, b1, True), (w2, b2, False),
              (dw0, db0, True), (dw1, db1, True), (dw2, db2, False)]
    for w, b, act in layers:
        h = jnp.dot(h, w[...],
                    preferred_element_type=jnp.float32) + b[...]
        if act:
            h = jnp.maximum(h, 0.0)
    o_ref[...] = h


def _head(g, p):
    names = ["cls_w0", "cls_b0", "cls_w1", "cls_b1", "cls_w2", "cls_b2",
             "dec_w0", "dec_b0", "dec_w1", "dec_b1", "dec_w2", "dec_b2"]
    args = []
    for n in names:
        a = p[n]
        args.append(a.reshape(1, -1) if a.ndim == 1 else a)
    return pl.pallas_call(
        _head_body,
        out_shape=jax.ShapeDtypeStruct((_B, 3 * _PS), jnp.float32),
    )(g, *args)


def kernel(pos, batch, params):
    N = pos.shape[0]
    batch = batch.astype(jnp.int32)
    idx, wn = _knn(pos, batch)
    nbr_flat = idx[:, 1:].T.reshape(-1)
    wn3 = wn[:, 1:].T.reshape(_K, N, 1)

    x = pos
    outs = []
    ci = 3
    for l, c in enumerate(_CONV):
        w0 = params[f"c{l}_w0"]
        wd = w0[:ci] - w0[ci:]
        wb = w0[ci:]
        u, v = _proj(x, wd, wb, params[f"c{l}_b0"].reshape(1, -1))
        if c < 128:
            v = jnp.pad(v, ((0, 0), (0, 128 - c)))
        vg = _gather_rows(v, nbr_flat).reshape(_K, N, v.shape[-1])
        x = _edge(u, vg, wn3, params[f"c{l}_w1"], params[f"c{l}_b1"].reshape(1, -1))
        outs.append(x)
        ci = c

    cat = jnp.concatenate(outs, axis=-1)
    m, s = _pool(cat, params["emb_w"], params["emb_b"].reshape(1, -1),
                 batch.reshape(N, 1))
    cnt = jnp.sum((batch[:, None] == jnp.arange(_B)[None, :]).astype(jnp.float32),
                  axis=0)
    g = jnp.concatenate([m, s / cnt[:, None]], axis=-1)
    out = _head(g, params)
    return out.reshape(-1, _PS, 3)


# speedup vs baseline: 130.6417x; 1.0052x over previous
import functools

import jax
import jax.numpy as jnp
from jax import lax
from jax.experimental import pallas as pl
from jax.experimental.pallas import tpu as pltpu
from jax.experimental.pallas import tpu_sc as plsc

_CONV = [64, 64, 128, 256]
_K = 20
_REG = 1e-3
_B = 8
_PS = 1024
_BIG = 3e38


def _knn_extract(co, posr_ref, post_ref, batr_ref, batt_ref, d2_ref):
    P, W = d2_ref.shape
    acc = jnp.zeros((P, W), jnp.float32)
    for c in range(3):
        xi = posr_ref[:, c:c + 1]
        xj = post_ref[c:c + 1, pl.ds(co, W)]
        d = xi - xj
        acc = acc + d * d
    same = batr_ref[...] == batt_ref[:, pl.ds(co, W)]
    d2_ref[...] = jnp.where(same, acc, jnp.full((P, W), _BIG, jnp.float32))
    cols = lax.broadcasted_iota(jnp.int32, (P, W), 1)
    vals, idxs = [], []
    for k in range(_K + 1):
        buf = d2_ref[...]
        m = jnp.min(buf, axis=1, keepdims=True)
        am = jnp.min(jnp.where(buf == m, cols, jnp.int32(W)), axis=1, keepdims=True)
        idxs.append(am + co)
        vals.append(m)
        d2_ref[...] = jnp.where(cols == am, jnp.float32(jnp.inf), buf)
    w = [jnp.exp(-v) for v in vals[1:]]
    ws = functools.reduce(lambda a, b: a + b, w)
    inv = 1.0 / (ws + jnp.float32(_REG))
    wcols = [jnp.zeros_like(inv)] + [wk * inv for wk in w]
    return jnp.concatenate(idxs, axis=1), jnp.concatenate(wcols, axis=1)


def _knn_body(coa_ref, cob_ref, posr_ref, post_ref, batr_ref, batt_ref,
              idx_ref, wn_ref, d2_ref):
    i = pl.program_id(0)
    co_a = pl.multiple_of(coa_ref[i], 128)
    co_b = pl.multiple_of(cob_ref[i], 128)
    idx_a, wn_a = _knn_extract(co_a, posr_ref, post_ref, batr_ref, batt_ref,
                               d2_ref)
    straddle = co_b != co_a

    @pl.when(straddle)
    def _():
        idx_b, wn_b = _knn_extract(co_b, posr_ref, post_ref, batr_ref,
                                   batt_ref, d2_ref)
        rm = batr_ref[...] == batr_ref[0:1, :]
        idx_ref[...] = jnp.where(rm, idx_a, idx_b)
        wn_ref[...] = jnp.where(rm, wn_a, wn_b)

    @pl.when(jnp.logical_not(straddle))
    def _():
        idx_ref[...] = idx_a
        wn_ref[...] = wn_a


_KNN_W = 1408


def _knn(pos, batch):
    N = pos.shape[0]
    P = 128
    nt = N // P
    posT = pos.T
    bat_r = batch.reshape(N, 1)
    bat_t = batch.reshape(1, N)

    starts = jnp.searchsorted(batch, jnp.arange(_B + 1, dtype=jnp.int32)
                              ).astype(jnp.int32)
    win = min(_KNN_W, N)
    cpc = jnp.minimum((starts[:-1] // 128) * 128, N - win)
    co_a = cpc[batch[::P]]
    co_b = cpc[batch[P - 1::P]]
    sz = starts[1:] - starts[:-1]
    fits = (jnp.max(starts[1:] - cpc) <= win) & (jnp.min(sz) >= P)

    def run(width, ca, cb):
        grid_spec = pltpu.PrefetchScalarGridSpec(
            num_scalar_prefetch=2,
            grid=(nt,),
            in_specs=[
                pl.BlockSpec((P, 3), lambda i, s1, s2: (i, 0)),
                pl.BlockSpec((3, N), lambda i, s1, s2: (0, 0)),
                pl.BlockSpec((P, 1), lambda i, s1, s2: (i, 0)),
                pl.BlockSpec((1, N), lambda i, s1, s2: (0, 0)),
            ],
            out_specs=[
                pl.BlockSpec((P, _K + 1), lambda i, s1, s2: (i, 0)),
                pl.BlockSpec((P, _K + 1), lambda i, s1, s2: (i, 0)),
            ],
            scratch_shapes=[pltpu.VMEM((P, width), jnp.float32)],
        )
        return pl.pallas_call(
            _knn_body,
            grid_spec=grid_spec,
            out_shape=[
                jax.ShapeDtypeStruct((N, _K + 1), jnp.int32),
                jax.ShapeDtypeStruct((N, _K + 1), jnp.float32),
            ],
        )(ca, cb, pos, posT, bat_r, bat_t)

    zeros = jnp.zeros((nt,), jnp.int32)
    if win == N:
        return run(N, zeros, zeros)
    return lax.cond(
        fits,
        lambda: run(win, co_a, co_b),
        lambda: run(N, zeros, zeros),
    )


def _gather_rows(table, idx):
    V, D = table.shape
    M = idx.shape[0]
    info = plsc.get_sparse_core_info()
    NC = info.num_cores
    NW = NC * info.num_subcores
    per_w = M // NW
    CH = 128
    n_ch = per_w // CH
    nbuf = 4 if D <= 128 else 2
    idx2 = idx.reshape(M // CH, CH)
    mesh = plsc.VectorSubcoreMesh(core_axis_name="c", subcore_axis_name="s")

    @functools.partial(
        pl.kernel,
        mesh=mesh,
        out_type=jax.ShapeDtypeStruct((M, D), jnp.float32),
        scratch_types=(
            [pltpu.VMEM((n_ch, CH), jnp.int32)]
            + [pltpu.VMEM((CH, D), jnp.float32)] * nbuf
            + [pltpu.SemaphoreType.DMA] * (2 * nbuf)
        ),
    )
    def gk(table_hbm, idx_hbm, out_hbm, idx_v, *bufs_sems):
        rows = bufs_sems[:nbuf]
        gsem = bufs_sems[nbuf:2 * nbuf]
        ssem = bufs_sems[2 * nbuf:]
        wid = lax.axis_index("s") * NC + lax.axis_index("c")
        base = wid * per_w
        pltpu.sync_copy(idx_hbm.at[pl.ds(wid * n_ch, n_ch)], idx_v)

        def body(j, _):
            copies = []
            for b in range(nbuf):
                copies.append(pltpu.async_copy(
                    table_hbm.at[idx_v.at[nbuf * j + b]], rows[b], gsem[b]))
            stores = []
            for b in range(nbuf):
                copies[b].wait()
                stores.append(pltpu.async_copy(
                    rows[b], out_hbm.at[pl.ds(base + (nbuf * j + b) * CH, CH)],
                    ssem[b]))
            for b in range(nbuf):
                stores[b].wait()
            return _

        lax.fori_loop(0, n_ch // nbuf, body, 0)

    return gk(table, idx2)


def _proj_body(x_ref, wd_ref, wb_ref, b0_ref, u_ref, v_ref):
    x = x_ref[...]
    u_ref[...] = jnp.dot(x, wd_ref[...],
                         preferred_element_type=jnp.float32) + b0_ref[...]
    v_ref[...] = jnp.dot(x, wb_ref[...],
                         preferred_element_type=jnp.float32)


def _proj(x, wd, wb, b0):
    N, ci = x.shape
    c = wd.shape[1]
    P = 512
    return pl.pallas_call(
        _proj_body,
        grid=(N // P,),
        in_specs=[
            pl.BlockSpec((P, ci), lambda i: (i, 0)),
            pl.BlockSpec((ci, c), lambda i: (0, 0)),
            pl.BlockSpec((ci, c), lambda i: (0, 0)),
            pl.BlockSpec((1, c), lambda i: (0, 0)),
        ],
        out_specs=[
            pl.BlockSpec((P, c), lambda i: (i, 0)),
            pl.BlockSpec((P, c), lambda i: (i, 0)),
        ],
        out_shape=[
            jax.ShapeDtypeStruct((N, c), jnp.float32),
            jax.ShapeDtypeStruct((N, c), jnp.float32),
        ],
    )(x, wd, wb, b0)


def _edge_body(u_ref, vg_ref, wn_ref, w1_ref, b1_ref, o_ref):
    P, c = u_ref.shape
    vg = vg_ref[...][:, :, :c]
    h1 = jnp.maximum(u_ref[...][None] + vg, 0.0)
    h2f = jnp.maximum(
        jnp.dot(h1.reshape(_K * P, c), w1_ref[...],
                preferred_element_type=jnp.float32) + b1_ref[...], 0.0)
    h2 = h2f.reshape(_K, P, c)
    mx = jnp.max(h2, axis=0)
    mn = jnp.sum(wn_ref[...] * h2, axis=0)
    o_ref[...] = mx + mn


def _edge(u, vg, wn3, w1, b1):
    N, c = u.shape
    Dp = vg.shape[-1]
    P = 256
    return pl.pallas_call(
        _edge_body,
        grid=(N // P,),
        in_specs=[
            pl.BlockSpec((P, c), lambda i: (i, 0)),
            pl.BlockSpec((_K, P, Dp), lambda i: (0, i, 0)),
            pl.BlockSpec((_K, P, 1), lambda i: (0, i, 0)),
            pl.BlockSpec((c, c), lambda i: (0, 0)),
            pl.BlockSpec((1, c), lambda i: (0, 0)),
        ],
        out_specs=pl.BlockSpec((P, c), lambda i: (i, 0)),
        out_shape=jax.ShapeDtypeStruct((N, c), jnp.float32),
    )(u, vg, wn3, w1, b1)


def _pool_body(cat_ref, ew_ref, eb_ref, bat_ref, m_ref, s_ref):
    i = pl.program_id(0)
    e = jnp.maximum(
        jnp.dot(cat_ref[...], ew_ref[...],
                preferred_element_type=jnp.float32) + eb_ref[...], 0.0)

    @pl.when(i == 0)
    def _():
        m_ref[...] = jnp.full(m_ref.shape, -jnp.inf, jnp.float32)
        s_ref[...] = jnp.zeros(s_ref.shape, jnp.float32)

    bat = bat_ref[...]
    oh = (bat == lax.broadcasted_iota(jnp.int32, (1, _B), 1)).astype(jnp.float32)
    s_ref[...] = s_ref[...] + lax.dot_general(
        oh, e, (((0,), (0,)), ((), ())), precision="highest",
        preferred_element_type=jnp.float32)
    for b in range(_B):
        mb = bat == b
        m_ref[b:b + 1, :] = jnp.maximum(
            m_ref[b:b + 1, :],
            jnp.max(jnp.where(mb, e, -jnp.inf), axis=0, keepdims=True))


def _pool(cat, ew, eb, bat):
    N, ci = cat.shape
    L = ew.shape[1]
    P = 512
    return pl.pallas_call(
        _pool_body,
        grid=(N // P,),
        in_specs=[
            pl.BlockSpec((P, ci), lambda i: (i, 0)),
            pl.BlockSpec((ci, L), lambda i: (0, 0)),
            pl.BlockSpec((1, L), lambda i: (0, 0)),
            pl.BlockSpec((P, 1), lambda i: (i, 0)),
        ],
        out_specs=[
            pl.BlockSpec((_B, L), lambda i: (0, 0)),
            pl.BlockSpec((_B, L), lambda i: (0, 0)),
        ],
        out_shape=[
            jax.ShapeDtypeStruct((_B, L), jnp.float32),
            jax.ShapeDtypeStruct((_B, L), jnp.float32),
        ],
    )(cat, ew, eb, bat)


def _head_body(g_ref, w0, b0, w1, b1, w2, b2, dw0, db0, dw1, db1, dw2, db2,
               o_ref):
    h = g_ref[...]
    layers = [(w0, b0, True), (w1, b1, True), (w2, b2, False),
              (dw0, db0, True), (dw1, db1, True), (dw2, db2, False)]
    for w, b, act in layers:
        h = jnp.dot(h, w[...],
                    preferred_element_type=jnp.float32) + b[...]
        if act:
            h = jnp.maximum(h, 0.0)
    o_ref[...] = h


def _head(g, p):
    names = ["cls_w0", "cls_b0", "cls_w1", "cls_b1", "cls_w2", "cls_b2",
             "dec_w0", "dec_b0", "dec_w1", "dec_b1", "dec_w2", "dec_b2"]
    args = []
    for n in names:
        a = p[n]
        args.append(a.reshape(1, -1) if a.ndim == 1 else a)
    return pl.pallas_call(
        _head_body,
        out_shape=jax.ShapeDtypeStruct((_B, 3 * _PS), jnp.float32),
    )(g, *args)


def kernel(pos, batch, params):
    N = pos.shape[0]
    batch = batch.astype(jnp.int32)
    idx, wn = _knn(pos, batch)
    nbr_flat = idx[:, 1:].T.reshape(-1)
    wn3 = wn[:, 1:].T.reshape(_K, N, 1)

    x = pos
    outs = []
    ci = 3
    for l, c in enumerate(_CONV):
        w0 = params[f"c{l}_w0"]
        wd = w0[:ci] - w0[ci:]
        wb = w0[ci:]
        u, v = _proj(x, wd, wb, params[f"c{l}_b0"].reshape(1, -1))
        if c < 128:
            v = jnp.pad(v, ((0, 0), (0, 128 - c)))
        vg = _gather_rows(v, nbr_flat).reshape(_K, N, v.shape[-1])
        x = _edge(u, vg, wn3, params[f"c{l}_w1"], params[f"c{l}_b1"].reshape(1, -1))
        outs.append(x)
        ci = c

    cat = jnp.concatenate(outs, axis=-1)
    m, s = _pool(cat, params["emb_w"], params["emb_b"].reshape(1, -1),
                 batch.reshape(N, 1))
    cnt = jnp.sum((batch[:, None] == jnp.arange(_B)[None, :]).astype(jnp.float32),
                  axis=0)
    g = jnp.concatenate([m, s / cnt[:, None]], axis=-1)
    out = _head(g, params)
    return out.reshape(-1, _PS, 3)


# speedup vs baseline: 132.0535x; 1.0108x over previous
import functools

import jax
import jax.numpy as jnp
from jax import lax
from jax.experimental import pallas as pl
from jax.experimental.pallas import tpu as pltpu
from jax.experimental.pallas import tpu_sc as plsc

_CONV = [64, 64, 128, 256]
_K = 20
_REG = 1e-3
_B = 8
_PS = 1024
_BIG = 3e38


def _knn_extract(co, posr_ref, post_ref, batr_ref, batt_ref, d2_ref):
    P, W = d2_ref.shape
    acc = jnp.zeros((P, W), jnp.float32)
    for c in range(3):
        xi = posr_ref[:, c:c + 1]
        xj = post_ref[c:c + 1, pl.ds(co, W)]
        d = xi - xj
        acc = acc + d * d
    same = batr_ref[...] == batt_ref[:, pl.ds(co, W)]
    buf = jnp.where(same, acc, jnp.full((P, W), _BIG, jnp.float32))
    cols = lax.broadcasted_iota(jnp.int32, (P, W), 1)
    vals, idxs = [], []
    for k in range(_K + 1):
        m = jnp.min(buf, axis=1, keepdims=True)
        am = jnp.min(jnp.where(buf == m, cols, jnp.int32(W)), axis=1, keepdims=True)
        idxs.append(am + co)
        vals.append(m)
        buf = jnp.where(cols == am, jnp.float32(jnp.inf), buf)
    w = [jnp.exp(-v) for v in vals[1:]]
    ws = functools.reduce(lambda a, b: a + b, w)
    inv = 1.0 / (ws + jnp.float32(_REG))
    wcols = [jnp.zeros_like(inv)] + [wk * inv for wk in w]
    return jnp.concatenate(idxs, axis=1), jnp.concatenate(wcols, axis=1)


def _knn_body(coa_ref, cob_ref, posr_ref, post_ref, batr_ref, batt_ref,
              idx_ref, wn_ref, d2_ref):
    i = pl.program_id(0)
    co_a = pl.multiple_of(coa_ref[i], 128)
    co_b = pl.multiple_of(cob_ref[i], 128)
    idx_a, wn_a = _knn_extract(co_a, posr_ref, post_ref, batr_ref, batt_ref,
                               d2_ref)
    straddle = co_b != co_a

    @pl.when(straddle)
    def _():
        idx_b, wn_b = _knn_extract(co_b, posr_ref, post_ref, batr_ref,
                                   batt_ref, d2_ref)
        rm = batr_ref[...] == batr_ref[0:1, :]
        idx_ref[...] = jnp.where(rm, idx_a, idx_b)
        wn_ref[...] = jnp.where(rm, wn_a, wn_b)

    @pl.when(jnp.logical_not(straddle))
    def _():
        idx_ref[...] = idx_a
        wn_ref[...] = wn_a


_KNN_W = 1408


def _knn(pos, batch):
    N = pos.shape[0]
    P = 128
    nt = N // P
    posT = pos.T
    bat_r = batch.reshape(N, 1)
    bat_t = batch.reshape(1, N)

    starts = jnp.searchsorted(batch, jnp.arange(_B + 1, dtype=jnp.int32)
                              ).astype(jnp.int32)
    win = min(_KNN_W, N)
    cpc = jnp.minimum((starts[:-1] // 128) * 128, N - win)
    co_a = cpc[batch[::P]]
    co_b = cpc[batch[P - 1::P]]
    sz = starts[1:] - starts[:-1]
    fits = (jnp.max(starts[1:] - cpc) <= win) & (jnp.min(sz) >= P)

    def run(width, ca, cb):
        grid_spec = pltpu.PrefetchScalarGridSpec(
            num_scalar_prefetch=2,
            grid=(nt,),
            in_specs=[
                pl.BlockSpec((P, 3), lambda i, s1, s2: (i, 0)),
                pl.BlockSpec((3, N), lambda i, s1, s2: (0, 0)),
                pl.BlockSpec((P, 1), lambda i, s1, s2: (i, 0)),
                pl.BlockSpec((1, N), lambda i, s1, s2: (0, 0)),
            ],
            out_specs=[
                pl.BlockSpec((P, _K + 1), lambda i, s1, s2: (i, 0)),
                pl.BlockSpec((P, _K + 1), lambda i, s1, s2: (i, 0)),
            ],
            scratch_shapes=[pltpu.VMEM((P, width), jnp.float32)],
        )
        return pl.pallas_call(
            _knn_body,
            grid_spec=grid_spec,
            out_shape=[
                jax.ShapeDtypeStruct((N, _K + 1), jnp.int32),
                jax.ShapeDtypeStruct((N, _K + 1), jnp.float32),
            ],
        )(ca, cb, pos, posT, bat_r, bat_t)

    zeros = jnp.zeros((nt,), jnp.int32)
    if win == N:
        return run(N, zeros, zeros)
    return lax.cond(
        fits,
        lambda: run(win, co_a, co_b),
        lambda: run(N, zeros, zeros),
    )


def _gather_rows(table, idx):
    V, D = table.shape
    M = idx.shape[0]
    info = plsc.get_sparse_core_info()
    NC = info.num_cores
    NW = NC * info.num_subcores
    per_w = M // NW
    CH = 128
    n_ch = per_w // CH
    nbuf = 4 if D <= 128 else 2
    idx2 = idx.reshape(M // CH, CH)
    mesh = plsc.VectorSubcoreMesh(core_axis_name="c", subcore_axis_name="s")

    @functools.partial(
        pl.kernel,
        mesh=mesh,
        out_type=jax.ShapeDtypeStruct((M, D), jnp.float32),
        scratch_types=(
            [pltpu.VMEM((n_ch, CH), jnp.int32)]
            + [pltpu.VMEM((CH, D), jnp.float32)] * nbuf
            + [pltpu.SemaphoreType.DMA] * (2 * nbuf)
        ),
    )
    def gk(table_hbm, idx_hbm, out_hbm, idx_v, *bufs_sems):
        rows = bufs_sems[:nbuf]
        gsem = bufs_sems[nbuf:2 * nbuf]
        ssem = bufs_sems[2 * nbuf:]
        wid = lax.axis_index("s") * NC + lax.axis_index("c")
        base = wid * per_w
        pltpu.sync_copy(idx_hbm.at[pl.ds(wid * n_ch, n_ch)], idx_v)

        def body(j, _):
            copies = []
            for b in range(nbuf):
                copies.append(pltpu.async_copy(
                    table_hbm.at[idx_v.at[nbuf * j + b]], rows[b], gsem[b]))
            stores = []
            for b in range(nbuf):
                copies[b].wait()
                stores.append(pltpu.async_copy(
                    rows[b], out_hbm.at[pl.ds(base + (nbuf * j + b) * CH, CH)],
                    ssem[b]))
            for b in range(nbuf):
                stores[b].wait()
            return _

        lax.fori_loop(0, n_ch // nbuf, body, 0)

    return gk(table, idx2)


def _proj_body(x_ref, wd_ref, wb_ref, b0_ref, u_ref, v_ref):
    x = x_ref[...]
    u_ref[...] = jnp.dot(x, wd_ref[...],
                         preferred_element_type=jnp.float32) + b0_ref[...]
    v_ref[...] = jnp.dot(x, wb_ref[...],
                         preferred_element_type=jnp.float32)


def _proj(x, wd, wb, b0):
    N, ci = x.shape
    c = wd.shape[1]
    P = 512
    return pl.pallas_call(
        _proj_body,
        grid=(N // P,),
        in_specs=[
            pl.BlockSpec((P, ci), lambda i: (i, 0)),
            pl.BlockSpec((ci, c), lambda i: (0, 0)),
            pl.BlockSpec((ci, c), lambda i: (0, 0)),
            pl.BlockSpec((1, c), lambda i: (0, 0)),
        ],
        out_specs=[
            pl.BlockSpec((P, c), lambda i: (i, 0)),
            pl.BlockSpec((P, c), lambda i: (i, 0)),
        ],
        out_shape=[
            jax.ShapeDtypeStruct((N, c), jnp.float32),
            jax.ShapeDtypeStruct((N, c), jnp.float32),
        ],
    )(x, wd, wb, b0)


def _edge_body(u_ref, vg_ref, wn_ref, w1_ref, b1_ref, o_ref):
    P, c = u_ref.shape
    vg = vg_ref[...][:, :, :c]
    h1 = jnp.maximum(u_ref[...][None] + vg, 0.0)
    h2f = jnp.maximum(
        jnp.dot(h1.reshape(_K * P, c), w1_ref[...],
                preferred_element_type=jnp.float32) + b1_ref[...], 0.0)
    h2 = h2f.reshape(_K, P, c)
    mx = jnp.max(h2, axis=0)
    mn = jnp.sum(wn_ref[...] * h2, axis=0)
    o_ref[...] = mx + mn


def _edge(u, vg, wn3, w1, b1):
    N, c = u.shape
    Dp = vg.shape[-1]
    P = 256
    return pl.pallas_call(
        _edge_body,
        grid=(N // P,),
        in_specs=[
            pl.BlockSpec((P, c), lambda i: (i, 0)),
            pl.BlockSpec((_K, P, Dp), lambda i: (0, i, 0)),
            pl.BlockSpec((_K, P, 1), lambda i: (0, i, 0)),
            pl.BlockSpec((c, c), lambda i: (0, 0)),
            pl.BlockSpec((1, c), lambda i: (0, 0)),
        ],
        out_specs=pl.BlockSpec((P, c), lambda i: (i, 0)),
        out_shape=jax.ShapeDtypeStruct((N, c), jnp.float32),
    )(u, vg, wn3, w1, b1)


def _pool_body(cat_ref, ew_ref, eb_ref, bat_ref, m_ref, s_ref):
    i = pl.program_id(0)
    e = jnp.maximum(
        jnp.dot(cat_ref[...], ew_ref[...],
                preferred_element_type=jnp.float32) + eb_ref[...], 0.0)

    @pl.when(i == 0)
    def _():
        m_ref[...] = jnp.full(m_ref.shape, -jnp.inf, jnp.float32)
        s_ref[...] = jnp.zeros(s_ref.shape, jnp.float32)

    bat = bat_ref[...]
    oh = (bat == lax.broadcasted_iota(jnp.int32, (1, _B), 1)).astype(jnp.float32)
    s_ref[...] = s_ref[...] + lax.dot_general(
        oh, e, (((0,), (0,)), ((), ())), precision="highest",
        preferred_element_type=jnp.float32)
    for b in range(_B):
        mb = bat == b
        m_ref[b:b + 1, :] = jnp.maximum(
            m_ref[b:b + 1, :],
            jnp.max(jnp.where(mb, e, -jnp.inf), axis=0, keepdims=True))


def _pool(cat, ew, eb, bat):
    N, ci = cat.shape
    L = ew.shape[1]
    P = 512
    return pl.pallas_call(
        _pool_body,
        grid=(N // P,),
        in_specs=[
            pl.BlockSpec((P, ci), lambda i: (i, 0)),
            pl.BlockSpec((ci, L), lambda i: (0, 0)),
            pl.BlockSpec((1, L), lambda i: (0, 0)),
            pl.BlockSpec((P, 1), lambda i: (i, 0)),
        ],
        out_specs=[
            pl.BlockSpec((_B, L), lambda i: (0, 0)),
            pl.BlockSpec((_B, L), lambda i: (0, 0)),
        ],
        out_shape=[
            jax.ShapeDtypeStruct((_B, L), jnp.float32),
            jax.ShapeDtypeStruct((_B, L), jnp.float32),
        ],
    )(cat, ew, eb, bat)


def _head_body(g_ref, w0, b0, w1, b1, w2, b2, dw0, db0, dw1, db1, dw2, db2,
               o_ref):
    h = g_ref[...]
    layers = [(w0, b0, True), (w1, b1, True), (w2, b2, False),
              (dw0, db0, True), (dw1, db1, True), (dw2, db2, False)]
    for w, b, act in layers:
        h = jnp.dot(h, w[...],
                    preferred_element_type=jnp.float32) + b[...]
        if act:
            h = jnp.maximum(h, 0.0)
    o_ref[...] = h


def _head(g, p):
    names = ["cls_w0", "cls_b0", "cls_w1", "cls_b1", "cls_w2", "cls_b2",
             "dec_w0", "dec_b0", "dec_w1", "dec_b1", "dec_w2", "dec_b2"]
    args = []
    for n in names:
        a = p[n]
        args.append(a.reshape(1, -1) if a.ndim == 1 else a)
    return pl.pallas_call(
        _head_body,
        out_shape=jax.ShapeDtypeStruct((_B, 3 * _PS), jnp.float32),
    )(g, *args)


def kernel(pos, batch, params):
    N = pos.shape[0]
    batch = batch.astype(jnp.int32)
    idx, wn = _knn(pos, batch)
    nbr_flat = idx[:, 1:].T.reshape(-1)
    wn3 = wn[:, 1:].T.reshape(_K, N, 1)

    x = pos
    outs = []
    ci = 3
    for l, c in enumerate(_CONV):
        w0 = params[f"c{l}_w0"]
        wd = w0[:ci] - w0[ci:]
        wb = w0[ci:]
        u, v = _proj(x, wd, wb, params[f"c{l}_b0"].reshape(1, -1))
        if c < 128:
            v = jnp.pad(v, ((0, 0), (0, 128 - c)))
        vg = _gather_rows(v, nbr_flat).reshape(_K, N, v.shape[-1])
        x = _edge(u, vg, wn3, params[f"c{l}_w1"], params[f"c{l}_b1"].reshape(1, -1))
        outs.append(x)
        ci = c

    cat = jnp.concatenate(outs, axis=-1)
    m, s = _pool(cat, params["emb_w"], params["emb_b"].reshape(1, -1),
                 batch.reshape(N, 1))
    cnt = jnp.sum((batch[:, None] == jnp.arange(_B)[None, :]).astype(jnp.float32),
                  axis=0)
    g = jnp.concatenate([m, s / cnt[:, None]], axis=-1)
    out = _head(g, params)
    return out.reshape(-1, _PS, 3)
